# baseline (speedup 1.0000x reference)
import functools

import jax
import jax.numpy as jnp
from jax import lax
from jax.experimental import pallas as pl
from jax.experimental.pallas import tpu as pltpu
from jax.experimental.pallas import tpu_sc as plsc

N = 10000
R = 24
H = 128
E = 320000

NC = 2
NS = 16
NW = NC * NS
EP = 327680
PER_W = EP // NW
PER_T = EP // NS
SB = 2048
NSUB = SB // 128
N_SEG = R * N
N_SEG_PAD = 253440
CNT_PER_T = N_SEG_PAD // NS
ZB = 3168


def _norm_body(dst_hbm, rel_hbm, c0_hbm, c1_hbm, cnt_sh, rbuf, cidx, obuf,
               zbuf, sem):
    c = lax.axis_index("c")
    s = lax.axis_index("s")

    def z1(i, _):
        zbuf[pl.ds(i * 16, 16)] = jnp.zeros((16,), jnp.float32)
        return 0
    lax.fori_loop(0, ZB // 16, z1, 0)
    for q in range(8):
        obuf[pl.ds(q * 16, 16)] = jnp.ones((16,), jnp.float32)
    for q in range(5):
        pltpu.sync_copy(
            zbuf, cnt_sh.at[pl.ds(s * CNT_PER_T + q * ZB, ZB)])
    plsc.subcore_barrier()

    def count_iter(t, _):
        off = (c * NS + s) * PER_W + t * SB
        off2 = (c * NS + s) * (PER_W // 128) + t * NSUB
        pltpu.sync_copy(rel_hbm.at[pl.ds(off, SB)], rbuf)
        pltpu.sync_copy(dst_hbm.at[pl.ds(off2, NSUB)], cidx)

        def mk(k, _):
            j = k // 8
            l = (k % 8) * 16
            rv = rbuf[pl.ds(k * 16, 16)]
            cidx[j, pl.ds(l, 16)] = rv * N + cidx[j, pl.ds(l, 16)]
            return 0
        lax.fori_loop(0, SB // 16, mk, 0)

        descs = [
            pltpu.async_copy(obuf, cnt_sh.at[cidx.at[j]], sem, add=True)
            for j in range(NSUB)
        ]
        for d in descs:
            d.wait()
        return 0
    lax.fori_loop(0, PER_W // SB, count_iter, 0)
    plsc.subcore_barrier()

    for q in range(5):
        o = s * CNT_PER_T + q * ZB
        pltpu.sync_copy(cnt_sh.at[pl.ds(o, ZB)], zbuf)

        @pl.when(c == 0)
        def _():
            pltpu.sync_copy(zbuf, c0_hbm.at[pl.ds(o, ZB)])

        @pl.when(c == 1)
        def _():
            pltpu.sync_copy(zbuf, c1_hbm.at[pl.ds(o, ZB)])


_norm_call = functools.partial(
    pl.kernel,
    out_type=(jax.ShapeDtypeStruct((N_SEG_PAD,), jnp.float32),
              jax.ShapeDtypeStruct((N_SEG_PAD,), jnp.float32)),
    mesh=plsc.VectorSubcoreMesh(core_axis_name="c", subcore_axis_name="s"),
    scratch_types=[
        pltpu.VMEM_SHARED((N_SEG_PAD,), jnp.float32),
        pltpu.VMEM((SB,), jnp.int32),
        pltpu.VMEM((NSUB, 128), jnp.int32),
        pltpu.VMEM((128,), jnp.float32),
        pltpu.VMEM((ZB,), jnp.float32),
        pltpu.SemaphoreType.DMA,
    ],
)(_norm_body)


def _agg_body(y_hbm, src_hbm, dst_hbm, rel_hbm, c0_hbm, c1_hbm, out_hbm,
              acc_sh, rbuf, gidx, cidx, didx, cntv, cnt2,
              rows0, rows1, rowsf, sem0, sem1, semn, semn2):
    c = lax.axis_index("c")
    s = lax.axis_index("s")
    wid = s * NC + c
    rows = (rows0, rows1)
    sems = (sem0, sem1)

    def z2(i, _):
        for q in range(H // 16):
            rowsf[i, pl.ds(q * 16, 16)] = jnp.zeros((16,), jnp.float32)
        return 0
    lax.fori_loop(0, 128, z2, 0)
    for q in range(5):
        pltpu.sync_copy(rowsf.at[pl.ds(0, 125)],
                        acc_sh.at[pl.ds(s * (N // NS) + q * 125, 125)])
    plsc.subcore_barrier()

    def main_iter(t, _):
        off = wid * PER_W + t * SB
        off2 = wid * (PER_W // 128) + t * NSUB
        pltpu.sync_copy(src_hbm.at[pl.ds(off2, NSUB)], gidx)
        pltpu.sync_copy(dst_hbm.at[pl.ds(off2, NSUB)], didx)
        pltpu.sync_copy(rel_hbm.at[pl.ds(off, SB)], rbuf)

        def mk(k, _):
            j = k // 8
            l = (k % 8) * 16
            rv = rbuf[pl.ds(k * 16, 16)] * N
            gidx[j, pl.ds(l, 16)] = rv + gidx[j, pl.ds(l, 16)]
            cidx[j, pl.ds(l, 16)] = rv + didx[j, pl.ds(l, 16)]
            return 0
        lax.fori_loop(0, SB // 16, mk, 0)

        d0 = pltpu.async_copy(y_hbm.at[gidx.at[0]], rows0, sem0)
        ndescs = [
            pltpu.async_copy(c0_hbm.at[cidx.at[j]], cntv.at[j], semn)
            for j in range(NSUB)
        ] + [
            pltpu.async_copy(c1_hbm.at[cidx.at[j]], cnt2.at[j], semn2)
            for j in range(NSUB)
        ]
        for d in ndescs:
            d.wait()

        def nrm(k, _):
            j = k // 8
            l = (k % 8) * 16
            tot = cntv[j, pl.ds(l, 16)] + cnt2[j, pl.ds(l, 16)]
            inv = 1.0 / jnp.maximum(tot, 1.0)
            civ = cidx[j, pl.ds(l, 16)]
            cntv[j, pl.ds(l, 16)] = jnp.where(civ < N_SEG, inv, 0.0)
            return 0
        lax.fori_loop(0, SB // 16, nrm, 0)

        pend = d0
        for j in range(NSUB):
            p = j % 2
            if j + 1 < NSUB:
                nxt = pltpu.async_copy(y_hbm.at[gidx.at[j + 1]],
                                       rows[(j + 1) % 2], sems[(j + 1) % 2])
            pend.wait()
            buf = rows[p]

            def scale(g, _):
                norm16 = cntv[j, pl.ds(g * 16, 16)]
                for m in range(16):
                    sv = jnp.full((16,), norm16[m], jnp.float32)
                    e = g * 16 + m
                    for c2 in range(H // 32):
                        w32 = buf[e, pl.ds(c2 * 16, 16)]
                        a = jax.lax.bitcast_convert_type(
                            w32 << 16, jnp.float32)
                        b = jax.lax.bitcast_convert_type(
                            w32 & jnp.int32(-65536), jnp.float32)
                        rowsf[e, pl.ds(c2 * 32, 16)] = a * sv
                        rowsf[e, pl.ds(c2 * 32 + 16, 16)] = b * sv
                return 0
            lax.fori_loop(0, 8, scale, 0)

            pltpu.sync_copy(rowsf, acc_sh.at[didx.at[j]], add=True)
            if j + 1 < NSUB:
                pend = nxt
        return 0
    lax.fori_loop(0, PER_W // SB, main_iter, 0)
    plsc.subcore_barrier()

    r0 = pl.multiple_of(s * 640, 8)

    @pl.when(s < NS - 1)
    def _():
        pltpu.sync_copy(acc_sh.at[pl.ds(r0, 640)],
                        out_hbm.at[c, pl.ds(r0, 640)])

    @pl.when(s == NS - 1)
    def _():
        pltpu.sync_copy(acc_sh.at[pl.ds(9600, 400)],
                        out_hbm.at[c, pl.ds(9600, 400)])


_agg_call = functools.partial(
    pl.kernel,
    out_type=jax.ShapeDtypeStruct((NC, N, H), jnp.float32),
    mesh=plsc.VectorSubcoreMesh(core_axis_name="c", subcore_axis_name="s"),
    compiler_params=pltpu.CompilerParams(use_tc_tiling_on_sc=False),
    scratch_types=[
        pltpu.VMEM_SHARED((N, H), jnp.float32),
        pltpu.VMEM((SB,), jnp.int32),
        pltpu.VMEM((NSUB, 128), jnp.int32),
        pltpu.VMEM((NSUB, 128), jnp.int32),
        pltpu.VMEM((NSUB, 128), jnp.int32),
        pltpu.VMEM((NSUB, 128), jnp.float32),
        pltpu.VMEM((NSUB, 128), jnp.float32),
        pltpu.VMEM((128, H // 2), jnp.int32),
        pltpu.VMEM((128, H // 2), jnp.int32),
        pltpu.VMEM((128, H), jnp.float32),
        pltpu.SemaphoreType.DMA,
        pltpu.SemaphoreType.DMA,
        pltpu.SemaphoreType.DMA,
        pltpu.SemaphoreType.DMA,
    ],
)(_agg_body)


def _relmat_body(x_ref, w_ref, y_ref):
    yf = jnp.dot(x_ref[...], w_ref[0],
                 preferred_element_type=jnp.float32)
    ye = yf[:, :H // 2].astype(jnp.bfloat16)
    yo = yf[:, H // 2:].astype(jnp.bfloat16)
    ue = jax.lax.bitcast_convert_type(ye, jnp.uint16).astype(jnp.uint32)
    uo = jax.lax.bitcast_convert_type(yo, jnp.uint16).astype(jnp.uint32)
    y_ref[0] = jax.lax.bitcast_convert_type(ue | (uo << 16), jnp.int32)


def _final_body(p_ref, x_ref, root_ref, bias_ref, o_ref):
    x = x_ref[...]
    o_ref[...] = (p_ref[0] + p_ref[1]
                  + jnp.dot(x, root_ref[...],
                            preferred_element_type=jnp.float32)
                  + bias_ref[...][None, :] + x)


def kernel(node_embeds, weight, root, bias, edge_index, edge_type):
    x = node_embeds.astype(jnp.float32)

    kg = jnp.arange(H // 2, dtype=jnp.int32)
    pe = (kg // 16) * 32 + kg % 16
    po = pe + 16
    wf = weight.astype(jnp.float32)
    wp = jnp.concatenate([wf[:, :, pe], wf[:, :, po]], axis=2)
    y = pl.pallas_call(
        _relmat_body,
        grid=(R,),
        in_specs=[
            pl.BlockSpec((N, H), lambda r: (0, 0)),
            pl.BlockSpec((1, H, H), lambda r: (r, 0, 0)),
        ],
        out_specs=pl.BlockSpec((1, N, H // 2), lambda r: (r, 0, 0)),
        out_shape=jax.ShapeDtypeStruct((R, N, H // 2), jnp.int32),
    )(x, wp)
    y = y.reshape(R * N, H // 2)

    src = edge_index[0].astype(jnp.int32)
    dst = edge_index[1].astype(jnp.int32)
    rel = edge_type.astype(jnp.int32)
    pad = EP - E
    k = jnp.arange(pad, dtype=jnp.int32)
    src_pad = (k * 977) % 240000 - 240000
    dst_pad = 768 + (k % 9232)
    srcp = jnp.concatenate([src, src_pad]).reshape(EP // 128, 128)
    dstp = jnp.concatenate([dst, dst_pad]).reshape(EP // 128, 128)
    relp = jnp.pad(rel, (0, pad), constant_values=R)

    c0, c1 = _norm_call(dstp, relp)
    partials = _agg_call(y, srcp, dstp, relp, c0, c1)

    out = pl.pallas_call(
        _final_body,
        out_shape=jax.ShapeDtypeStruct((N, H), jnp.float32),
    )(partials, x, root.astype(jnp.float32), bias.astype(jnp.float32))
    return out


# speedup vs baseline: 2.0970x; 2.0970x over previous
import functools

import jax
import jax.numpy as jnp
from jax import lax
from jax.experimental import pallas as pl
from jax.experimental.pallas import tpu as pltpu
from jax.experimental.pallas import tpu_sc as plsc

N = 10000
R = 24
H = 128
E = 320000

NC = 2
NS = 16
NW = NC * NS
EP = 327680
PER_W = EP // NW
PER_T = EP // NS
SB = 2048
NSUB = SB // 128
N_SEG = R * N
N_SEG_PAD = 253440
CNT_PER_T = N_SEG_PAD // NS
ZB = 3168


def _norm_body(dst_hbm, rel_hbm, c0_hbm, c1_hbm, cnt_sh, rbuf, cidx, obuf,
               zbuf, sem):
    c = lax.axis_index("c")
    s = lax.axis_index("s")

    def z1(i, _):
        zbuf[pl.ds(i * 16, 16)] = jnp.zeros((16,), jnp.float32)
        return 0
    lax.fori_loop(0, ZB // 16, z1, 0)
    for q in range(8):
        obuf[pl.ds(q * 16, 16)] = jnp.ones((16,), jnp.float32)
    for q in range(5):
        pltpu.sync_copy(
            zbuf, cnt_sh.at[pl.ds(s * CNT_PER_T + q * ZB, ZB)])
    plsc.subcore_barrier()

    def count_iter(t, _):
        off = (c * NS + s) * PER_W + t * SB
        off2 = (c * NS + s) * (PER_W // 128) + t * NSUB
        pltpu.sync_copy(rel_hbm.at[pl.ds(off, SB)], rbuf)
        pltpu.sync_copy(dst_hbm.at[pl.ds(off2, NSUB)], cidx)

        def mk(k, _):
            j = k // 8
            l = (k % 8) * 16
            rv = rbuf[pl.ds(k * 16, 16)]
            cidx[j, pl.ds(l, 16)] = rv * N + cidx[j, pl.ds(l, 16)]
            return 0
        lax.fori_loop(0, SB // 16, mk, 0)

        descs = [
            pltpu.async_copy(obuf, cnt_sh.at[cidx.at[j]], sem, add=True)
            for j in range(NSUB)
        ]
        for d in descs:
            d.wait()
        return 0
    lax.fori_loop(0, PER_W // SB, count_iter, 0)
    plsc.subcore_barrier()

    for q in range(5):
        o = s * CNT_PER_T + q * ZB
        pltpu.sync_copy(cnt_sh.at[pl.ds(o, ZB)], zbuf)

        @pl.when(c == 0)
        def _():
            pltpu.sync_copy(zbuf, c0_hbm.at[pl.ds(o, ZB)])

        @pl.when(c == 1)
        def _():
            pltpu.sync_copy(zbuf, c1_hbm.at[pl.ds(o, ZB)])


_norm_call = functools.partial(
    pl.kernel,
    out_type=(jax.ShapeDtypeStruct((N_SEG_PAD,), jnp.float32),
              jax.ShapeDtypeStruct((N_SEG_PAD,), jnp.float32)),
    mesh=plsc.VectorSubcoreMesh(core_axis_name="c", subcore_axis_name="s"),
    scratch_types=[
        pltpu.VMEM_SHARED((N_SEG_PAD,), jnp.float32),
        pltpu.VMEM((SB,), jnp.int32),
        pltpu.VMEM((NSUB, 128), jnp.int32),
        pltpu.VMEM((128,), jnp.float32),
        pltpu.VMEM((ZB,), jnp.float32),
        pltpu.SemaphoreType.DMA,
    ],
)(_norm_body)


def _agg_body(y_hbm, src_hbm, dst_hbm, rel_hbm, c0_hbm, c1_hbm, out_hbm,
              acc_sh, rbuf, gidx, cidx, didx, cntv, cnt2,
              rows0, rows1, sem0, sem1, semn, semn2):
    c = lax.axis_index("c")
    s = lax.axis_index("s")
    wid = s * NC + c
    rows = (rows0, rows1)
    sems = (sem0, sem1)

    def z2(i, _):
        for q in range(H // 16):
            rows0[i, pl.ds(q * 16, 16)] = jnp.zeros((16,), jnp.float32)
        return 0
    lax.fori_loop(0, 128, z2, 0)
    for q in range(5):
        pltpu.sync_copy(rows0.at[pl.ds(0, 125)],
                        acc_sh.at[pl.ds(s * (N // NS) + q * 125, 125)])
    plsc.subcore_barrier()

    def main_iter(t, _):
        off = wid * PER_W + t * SB
        off2 = wid * (PER_W // 128) + t * NSUB
        pltpu.sync_copy(src_hbm.at[pl.ds(off2, NSUB)], gidx)
        pltpu.sync_copy(dst_hbm.at[pl.ds(off2, NSUB)], didx)
        pltpu.sync_copy(rel_hbm.at[pl.ds(off, SB)], rbuf)

        def mk(k, _):
            j = k // 8
            l = (k % 8) * 16
            rv = rbuf[pl.ds(k * 16, 16)] * N
            gidx[j, pl.ds(l, 16)] = rv + gidx[j, pl.ds(l, 16)]
            cidx[j, pl.ds(l, 16)] = rv + didx[j, pl.ds(l, 16)]
            return 0
        lax.fori_loop(0, SB // 16, mk, 0)

        d0 = pltpu.async_copy(y_hbm.at[gidx.at[0]], rows0, sem0)
        ndescs = [
            pltpu.async_copy(c0_hbm.at[cidx.at[j]], cntv.at[j], semn)
            for j in range(NSUB)
        ] + [
            pltpu.async_copy(c1_hbm.at[cidx.at[j]], cnt2.at[j], semn2)
            for j in range(NSUB)
        ]
        for d in ndescs:
            d.wait()

        def nrm(k, _):
            j = k // 8
            l = (k % 8) * 16
            tot = cntv[j, pl.ds(l, 16)] + cnt2[j, pl.ds(l, 16)]
            inv = 1.0 / jnp.maximum(tot, 1.0)
            civ = cidx[j, pl.ds(l, 16)]
            cntv[j, pl.ds(l, 16)] = jnp.where(civ < N_SEG, inv, 0.0)
            return 0
        lax.fori_loop(0, SB // 16, nrm, 0)

        pend = d0
        for j in range(NSUB):
            p = j % 2
            if j + 1 < NSUB:
                nxt = pltpu.async_copy(y_hbm.at[gidx.at[j + 1]],
                                       rows[(j + 1) % 2], sems[(j + 1) % 2])
            pend.wait()
            buf = rows[p]

            def scale(g, _):
                norm16 = cntv[j, pl.ds(g * 16, 16)]
                for m in range(16):
                    sv = jnp.full((16,), norm16[m], jnp.float32)
                    e = g * 16 + m
                    for q in range(H // 16):
                        buf[e, pl.ds(q * 16, 16)] = (
                            buf[e, pl.ds(q * 16, 16)] * sv)
                return 0
            lax.fori_loop(0, 8, scale, 0)

            pltpu.sync_copy(buf, acc_sh.at[didx.at[j]], add=True)
            if j + 1 < NSUB:
                pend = nxt
        return 0
    lax.fori_loop(0, PER_W // SB, main_iter, 0)
    plsc.subcore_barrier()

    r0 = pl.multiple_of(s * 640, 8)

    @pl.when(s < NS - 1)
    def _():
        pltpu.sync_copy(acc_sh.at[pl.ds(r0, 640)],
                        out_hbm.at[c, pl.ds(r0, 640)])

    @pl.when(s == NS - 1)
    def _():
        pltpu.sync_copy(acc_sh.at[pl.ds(9600, 400)],
                        out_hbm.at[c, pl.ds(9600, 400)])


_agg_call = functools.partial(
    pl.kernel,
    out_type=jax.ShapeDtypeStruct((NC, N, H), jnp.float32),
    mesh=plsc.VectorSubcoreMesh(core_axis_name="c", subcore_axis_name="s"),
    scratch_types=[
        pltpu.VMEM_SHARED((N, H), jnp.float32),
        pltpu.VMEM((SB,), jnp.int32),
        pltpu.VMEM((NSUB, 128), jnp.int32),
        pltpu.VMEM((NSUB, 128), jnp.int32),
        pltpu.VMEM((NSUB, 128), jnp.int32),
        pltpu.VMEM((NSUB, 128), jnp.float32),
        pltpu.VMEM((NSUB, 128), jnp.float32),
        pltpu.VMEM((128, H), jnp.float32),
        pltpu.VMEM((128, H), jnp.float32),
        pltpu.SemaphoreType.DMA,
        pltpu.SemaphoreType.DMA,
        pltpu.SemaphoreType.DMA,
        pltpu.SemaphoreType.DMA,
    ],
)(_agg_body)


def _relmat_body(x_ref, w_ref, y_ref):
    y_ref[0] = jnp.dot(x_ref[...], w_ref[0],
                       preferred_element_type=jnp.float32)


def _final_body(p_ref, x_ref, root_ref, bias_ref, o_ref):
    x = x_ref[...]
    o_ref[...] = (p_ref[0] + p_ref[1]
                  + jnp.dot(x, root_ref[...],
                            preferred_element_type=jnp.float32)
                  + bias_ref[...][None, :] + x)


def kernel(node_embeds, weight, root, bias, edge_index, edge_type):
    x = node_embeds.astype(jnp.float32)

    y = pl.pallas_call(
        _relmat_body,
        grid=(R,),
        in_specs=[
            pl.BlockSpec((N, H), lambda r: (0, 0)),
            pl.BlockSpec((1, H, H), lambda r: (r, 0, 0)),
        ],
        out_specs=pl.BlockSpec((1, N, H), lambda r: (r, 0, 0)),
        out_shape=jax.ShapeDtypeStruct((R, N, H), jnp.float32),
    )(x, weight.astype(jnp.float32))
    y = y.reshape(R * N, H)

    src = edge_index[0].astype(jnp.int32)
    dst = edge_index[1].astype(jnp.int32)
    rel = edge_type.astype(jnp.int32)
    pad = EP - E
    k = jnp.arange(pad, dtype=jnp.int32)
    src_pad = (k * 977) % 240000 - 240000
    dst_pad = 768 + (k % 9232)
    srcp = jnp.concatenate([src, src_pad]).reshape(EP // 128, 128)
    dstp = jnp.concatenate([dst, dst_pad]).reshape(EP // 128, 128)
    relp = jnp.pad(rel, (0, pad), constant_values=R)

    c0, c1 = _norm_call(dstp, relp)
    partials = _agg_call(y, srcp, dstp, relp, c0, c1)

    out = pl.pallas_call(
        _final_body,
        out_shape=jax.ShapeDtypeStruct((N, H), jnp.float32),
    )(partials, x, root.astype(jnp.float32), bias.astype(jnp.float32))
    return out
